# baseline (speedup 1.0000x reference)
import functools
import math

import jax
import jax.numpy as jnp
from jax import lax
from jax.experimental import pallas as pl
from jax.experimental.pallas import tpu as pltpu
from jax.experimental.pallas import tpu_sc as plsc

D_MODEL = 64
SCALE = math.sqrt(D_MODEL)

VOCAB = 1000000
NUM_WORKERS = 32
N_BATCH = 4096
N_TOK = 200
B_PER_W = N_BATCH // NUM_WORKERS
NBUF = 4
SPLIT = 104

PACK_W = 8192
PACK_GRID = (VOCAB + PACK_W - 1) // PACK_W
VOCAB_P = PACK_GRID * PACK_W


def _tc_pack_lut(lut):
    lut_t = jnp.transpose(lut)
    eye = jnp.eye(D_MODEL, dtype=jnp.float32)
    half = PACK_W // 2

    def body(in_ref, eye_ref, out_ref):
        dn = (((0,), (0,)), ((), ()))
        e = eye_ref[...]
        out_ref[:, 0:D_MODEL] = jax.lax.dot_general(
            in_ref[:, 0:half], e, dn,
            preferred_element_type=jnp.float32)
        out_ref[:, D_MODEL:2 * D_MODEL] = jax.lax.dot_general(
            in_ref[:, half:PACK_W], e, dn,
            preferred_element_type=jnp.float32)

    packed = pl.pallas_call(
        body,
        grid=(PACK_GRID,),
        in_specs=[pl.BlockSpec((D_MODEL, PACK_W), lambda i: (0, i)),
                  pl.BlockSpec((D_MODEL, D_MODEL), lambda i: (0, 0))],
        out_specs=pl.BlockSpec((half, 2 * D_MODEL), lambda i: (i, 0)),
        out_shape=jax.ShapeDtypeStruct((VOCAB_P // 2, 2 * D_MODEL),
                                       jnp.float32),
    )(lut_t, eye)
    return jnp.reshape(jnp.reshape(packed, (VOCAB_P * D_MODEL,)),
                       (VOCAB_P, D_MODEL))


def _permute_idx(x):
    return (jnp.left_shift(jnp.right_shift(x, 13), 13)
            + jnp.left_shift(jnp.bitwise_and(x, 4095), 1)
            + jnp.bitwise_and(jnp.right_shift(x, 12), 1))


def _tc_finish(d2):
    eye8 = jnp.eye(256, dtype=jnp.float32)

    def body(in_ref, eye_ref, out_ref):
        out_ref[...] = jax.lax.dot_general(
            in_ref[...], eye_ref[...], (((0,), (0,)), ((), ())),
            preferred_element_type=jnp.float32)

    out_t = pl.pallas_call(
        body,
        grid=(N_BATCH // 256, (N_TOK * D_MODEL) // 256),
        in_specs=[pl.BlockSpec((256, 256), lambda i, j: (i, j)),
                  pl.BlockSpec((256, 256), lambda i, j: (0, 0))],
        out_specs=pl.BlockSpec((256, 256), lambda i, j: (j, i)),
        out_shape=jax.ShapeDtypeStruct((N_TOK * D_MODEL, N_BATCH),
                                       jnp.float32),
    )(d2, eye8)
    out3 = jnp.reshape(out_t, (N_TOK, D_MODEL, N_BATCH))
    return jnp.transpose(out3, (2, 0, 1))


def _sc_embed(lut, x):
    mesh = plsc.VectorSubcoreMesh(core_axis_name="c", subcore_axis_name="s")
    info = plsc.get_sparse_core_info()
    nc = info.num_cores

    @functools.partial(
        pl.kernel,
        mesh=mesh,
        out_type=jax.ShapeDtypeStruct((N_BATCH, N_TOK * D_MODEL), jnp.float32),
        scratch_types=[
            pltpu.VMEM((B_PER_W, N_TOK), jnp.int32),
            pltpu.VMEM((NBUF, N_TOK, D_MODEL), jnp.float32),
            pltpu.VMEM((NBUF, N_TOK * D_MODEL), jnp.float32),
            pltpu.SemaphoreType.DMA((NBUF,)),
            pltpu.SemaphoreType.DMA((NBUF,)),
        ],
        compiler_params=pltpu.CompilerParams(use_tc_tiling_on_sc=False),
    )
    def k(lut_hbm, idx_hbm, out_hbm, idx_v, gbuf, sbuf, gsem, osem):
        wid = lax.axis_index("s") * nc + lax.axis_index("c")
        b0 = wid * B_PER_W
        pltpu.sync_copy(idx_hbm.at[pl.ds(b0, B_PER_W)], idx_v)

        def gather_start(s, i):
            pltpu.async_copy(lut_hbm.at[idx_v.at[i, pl.ds(0, SPLIT)]],
                             gbuf.at[s, pl.ds(0, SPLIT)], gsem.at[s])
            pltpu.async_copy(lut_hbm.at[idx_v.at[i, pl.ds(SPLIT, N_TOK - SPLIT)]],
                             gbuf.at[s, pl.ds(SPLIT, N_TOK - SPLIT)], gsem.at[s])

        def gather_wait(s, i):
            pltpu.make_async_copy(
                lut_hbm.at[idx_v.at[i, pl.ds(0, SPLIT)]],
                gbuf.at[s, pl.ds(0, SPLIT)], gsem.at[s]).wait()
            pltpu.make_async_copy(
                lut_hbm.at[idx_v.at[i, pl.ds(SPLIT, N_TOK - SPLIT)]],
                gbuf.at[s, pl.ds(SPLIT, N_TOK - SPLIT)], gsem.at[s]).wait()

        def out_start(s, i):
            pltpu.async_copy(sbuf.at[s], out_hbm.at[b0 + i], osem.at[s])

        def out_wait(s, i):
            pltpu.make_async_copy(sbuf.at[s], out_hbm.at[b0 + i],
                                  osem.at[s]).wait()

        for s in range(NBUF):
            gather_start(s, s)

        def body(it, carry):
            i0 = it * NBUF
            for s in range(NBUF):
                i = i0 + s

                @pl.when(i >= NBUF)
                def _():
                    out_wait(s, i - NBUF)

                gather_wait(s, i)

                def srow(r, c):
                    for q in range(D_MODEL // 16):
                        sbuf[s, pl.ds(r * D_MODEL + q * 16, 16)] = (
                            gbuf[s, r, pl.ds(q * 16, 16)] * SCALE)
                    return c

                lax.fori_loop(0, N_TOK, srow, 0)

                @pl.when(i + NBUF < B_PER_W)
                def _():
                    gather_start(s, i + NBUF)

                out_start(s, i)
            return carry

        lax.fori_loop(0, B_PER_W // NBUF, body, 0)

        for s in range(NBUF):
            out_wait(s, B_PER_W - NBUF + s)

    return k(lut, x)


def kernel(x, lut):
    lut_rm = _tc_pack_lut(lut)
    dense = _sc_embed(lut_rm, _permute_idx(x))
    return _tc_finish(dense)


# speedup vs baseline: 1.3918x; 1.3918x over previous
import functools
import math

import jax
import jax.numpy as jnp
from jax import lax
from jax.experimental import pallas as pl
from jax.experimental.pallas import tpu as pltpu
from jax.experimental.pallas import tpu_sc as plsc

D_MODEL = 64
SCALE = math.sqrt(D_MODEL)

VOCAB = 1000000
NUM_WORKERS = 32
N_BATCH = 4096
N_TOK = 200
B_PER_W = N_BATCH // NUM_WORKERS
NBUF = 4
SPLIT = 104

PACK_W = 8192
PACK_GRID = (VOCAB + PACK_W - 1) // PACK_W
VOCAB_P = PACK_GRID * PACK_W


def _tc_pack_lut(lut):
    lut_t = jnp.transpose(lut)
    eye = jnp.eye(D_MODEL, dtype=jnp.float32)
    half = PACK_W // 2

    def body(in_ref, eye_ref, out_ref):
        dn = (((0,), (0,)), ((), ()))
        e = eye_ref[...]
        out_ref[:, 0:D_MODEL] = jax.lax.dot_general(
            in_ref[:, 0:half], e, dn,
            preferred_element_type=jnp.float32)
        out_ref[:, D_MODEL:2 * D_MODEL] = jax.lax.dot_general(
            in_ref[:, half:PACK_W], e, dn,
            preferred_element_type=jnp.float32)

    packed = pl.pallas_call(
        body,
        grid=(PACK_GRID,),
        in_specs=[pl.BlockSpec((D_MODEL, PACK_W), lambda i: (0, i)),
                  pl.BlockSpec((D_MODEL, D_MODEL), lambda i: (0, 0))],
        out_specs=pl.BlockSpec((half, 2 * D_MODEL), lambda i: (i, 0)),
        out_shape=jax.ShapeDtypeStruct((VOCAB_P // 2, 2 * D_MODEL),
                                       jnp.float32),
    )(lut_t, eye)
    return jnp.reshape(jnp.reshape(packed, (VOCAB_P * D_MODEL,)),
                       (VOCAB_P, D_MODEL))


def _permute_idx(x):
    return (jnp.left_shift(jnp.right_shift(x, 13), 13)
            + jnp.left_shift(jnp.bitwise_and(x, 4095), 1)
            + jnp.bitwise_and(jnp.right_shift(x, 12), 1))


def _tc_finish(d2):
    def body(in_ref, out_ref):
        out_ref[...] = jnp.transpose(in_ref[...])

    out_t = pl.pallas_call(
        body,
        grid=(N_BATCH // 512, (N_TOK * D_MODEL) // 512),
        in_specs=[pl.BlockSpec((512, 512), lambda i, j: (i, j))],
        out_specs=pl.BlockSpec((512, 512), lambda i, j: (j, i)),
        out_shape=jax.ShapeDtypeStruct((N_TOK * D_MODEL, N_BATCH),
                                       jnp.float32),
    )(d2)
    out3 = jnp.reshape(out_t, (N_TOK, D_MODEL, N_BATCH))
    return jnp.transpose(out3, (2, 0, 1))


def _sc_embed(lut, x):
    mesh = plsc.VectorSubcoreMesh(core_axis_name="c", subcore_axis_name="s")
    info = plsc.get_sparse_core_info()
    nc = info.num_cores

    @functools.partial(
        pl.kernel,
        mesh=mesh,
        out_type=jax.ShapeDtypeStruct((N_BATCH, N_TOK * D_MODEL), jnp.float32),
        scratch_types=[
            pltpu.VMEM((B_PER_W, N_TOK), jnp.int32),
            pltpu.VMEM((NBUF, N_TOK, D_MODEL), jnp.float32),
            pltpu.VMEM((NBUF, N_TOK * D_MODEL), jnp.float32),
            pltpu.SemaphoreType.DMA((NBUF,)),
            pltpu.SemaphoreType.DMA((NBUF,)),
        ],
        compiler_params=pltpu.CompilerParams(use_tc_tiling_on_sc=False),
    )
    def k(lut_hbm, idx_hbm, out_hbm, idx_v, gbuf, sbuf, gsem, osem):
        wid = lax.axis_index("s") * nc + lax.axis_index("c")
        b0 = wid * B_PER_W
        pltpu.sync_copy(idx_hbm.at[pl.ds(b0, B_PER_W)], idx_v)

        def gather_start(s, i):
            pltpu.async_copy(lut_hbm.at[idx_v.at[i, pl.ds(0, SPLIT)]],
                             gbuf.at[s, pl.ds(0, SPLIT)], gsem.at[s])
            pltpu.async_copy(lut_hbm.at[idx_v.at[i, pl.ds(SPLIT, N_TOK - SPLIT)]],
                             gbuf.at[s, pl.ds(SPLIT, N_TOK - SPLIT)], gsem.at[s])

        def gather_wait(s, i):
            pltpu.make_async_copy(
                lut_hbm.at[idx_v.at[i, pl.ds(0, SPLIT)]],
                gbuf.at[s, pl.ds(0, SPLIT)], gsem.at[s]).wait()
            pltpu.make_async_copy(
                lut_hbm.at[idx_v.at[i, pl.ds(SPLIT, N_TOK - SPLIT)]],
                gbuf.at[s, pl.ds(SPLIT, N_TOK - SPLIT)], gsem.at[s]).wait()

        def out_start(s, i):
            pltpu.async_copy(sbuf.at[s], out_hbm.at[b0 + i], osem.at[s])

        def out_wait(s, i):
            pltpu.make_async_copy(sbuf.at[s], out_hbm.at[b0 + i],
                                  osem.at[s]).wait()

        for s in range(NBUF):
            gather_start(s, s)

        def body(it, carry):
            i0 = it * NBUF
            for s in range(NBUF):
                i = i0 + s

                @pl.when(i >= NBUF)
                def _():
                    out_wait(s, i - NBUF)

                gather_wait(s, i)

                def srow(r, c):
                    for q in range(D_MODEL // 16):
                        sbuf[s, pl.ds(r * D_MODEL + q * 16, 16)] = (
                            gbuf[s, r, pl.ds(q * 16, 16)] * SCALE)
                    return c

                lax.fori_loop(0, N_TOK, srow, 0)

                @pl.when(i + NBUF < B_PER_W)
                def _():
                    gather_start(s, i + NBUF)

                out_start(s, i)
            return carry

        lax.fori_loop(0, B_PER_W // NBUF, body, 0)

        for s in range(NBUF):
            out_wait(s, B_PER_W - NBUF + s)

    return k(lut, x)


def kernel(x, lut):
    lut_rm = _tc_pack_lut(lut)
    dense = _sc_embed(lut_rm, _permute_idx(x))
    return _tc_finish(dense)


# speedup vs baseline: 1.4474x; 1.0400x over previous
import functools
import math

import jax
import jax.numpy as jnp
from jax import lax
from jax.experimental import pallas as pl
from jax.experimental.pallas import tpu as pltpu
from jax.experimental.pallas import tpu_sc as plsc

D_MODEL = 64
SCALE = math.sqrt(D_MODEL)

VOCAB = 1000000
NUM_WORKERS = 32
N_BATCH = 4096
N_TOK = 200
B_PER_W = N_BATCH // NUM_WORKERS
NBUF = 4
SPLIT = 104

PACK_W = 16384
PACK_GRID = (VOCAB + PACK_W - 1) // PACK_W
VOCAB_P = PACK_GRID * PACK_W


def _tc_pack_lut(lut):
    lut_t = jnp.transpose(lut)
    eye = jnp.eye(D_MODEL, dtype=jnp.float32)
    half = PACK_W // 2

    def body(in_ref, eye_ref, out_ref):
        dn = (((0,), (0,)), ((), ()))
        e = eye_ref[...]
        out_ref[:, 0:D_MODEL] = jax.lax.dot_general(
            in_ref[:, 0:half], e, dn,
            preferred_element_type=jnp.float32)
        out_ref[:, D_MODEL:2 * D_MODEL] = jax.lax.dot_general(
            in_ref[:, half:PACK_W], e, dn,
            preferred_element_type=jnp.float32)

    packed = pl.pallas_call(
        body,
        grid=(PACK_GRID,),
        in_specs=[pl.BlockSpec((D_MODEL, PACK_W), lambda i: (0, i)),
                  pl.BlockSpec((D_MODEL, D_MODEL), lambda i: (0, 0))],
        out_specs=pl.BlockSpec((half, 2 * D_MODEL), lambda i: (i, 0)),
        out_shape=jax.ShapeDtypeStruct((VOCAB_P // 2, 2 * D_MODEL),
                                       jnp.float32),
    )(lut_t, eye)
    return jnp.reshape(jnp.reshape(packed, (VOCAB_P * D_MODEL,)),
                       (VOCAB_P, D_MODEL))


def _permute_idx(x):
    hbits = 13
    return (jnp.left_shift(jnp.right_shift(x, hbits + 1), hbits + 1)
            + jnp.left_shift(jnp.bitwise_and(x, (1 << hbits) - 1), 1)
            + jnp.bitwise_and(jnp.right_shift(x, hbits), 1))


def _tc_finish(d2):
    def body(in_ref, out_ref):
        out_ref[...] = jnp.transpose(in_ref[...])

    out_t = pl.pallas_call(
        body,
        grid=(N_BATCH // 512, (N_TOK * D_MODEL) // 512),
        in_specs=[pl.BlockSpec((512, 512), lambda i, j: (i, j))],
        out_specs=pl.BlockSpec((512, 512), lambda i, j: (j, i)),
        out_shape=jax.ShapeDtypeStruct((N_TOK * D_MODEL, N_BATCH),
                                       jnp.float32),
    )(d2)
    out3 = jnp.reshape(out_t, (N_TOK, D_MODEL, N_BATCH))
    return jnp.transpose(out3, (2, 0, 1))


def _sc_embed(lut, x):
    mesh = plsc.VectorSubcoreMesh(core_axis_name="c", subcore_axis_name="s")
    info = plsc.get_sparse_core_info()
    nc = info.num_cores

    @functools.partial(
        pl.kernel,
        mesh=mesh,
        out_type=jax.ShapeDtypeStruct((N_BATCH, N_TOK * D_MODEL), jnp.float32),
        scratch_types=[
            pltpu.VMEM((B_PER_W, N_TOK), jnp.int32),
            pltpu.VMEM((NBUF, N_TOK, D_MODEL), jnp.float32),
            pltpu.VMEM((NBUF, N_TOK * D_MODEL), jnp.float32),
            pltpu.SemaphoreType.DMA((NBUF,)),
            pltpu.SemaphoreType.DMA((NBUF,)),
        ],
        compiler_params=pltpu.CompilerParams(use_tc_tiling_on_sc=False),
    )
    def k(lut_hbm, idx_hbm, out_hbm, idx_v, gbuf, sbuf, gsem, osem):
        wid = lax.axis_index("s") * nc + lax.axis_index("c")
        b0 = wid * B_PER_W
        pltpu.sync_copy(idx_hbm.at[pl.ds(b0, B_PER_W)], idx_v)

        def gather_start(s, i):
            pltpu.async_copy(lut_hbm.at[idx_v.at[i, pl.ds(0, SPLIT)]],
                             gbuf.at[s, pl.ds(0, SPLIT)], gsem.at[s])
            pltpu.async_copy(lut_hbm.at[idx_v.at[i, pl.ds(SPLIT, N_TOK - SPLIT)]],
                             gbuf.at[s, pl.ds(SPLIT, N_TOK - SPLIT)], gsem.at[s])

        def gather_wait(s, i):
            pltpu.make_async_copy(
                lut_hbm.at[idx_v.at[i, pl.ds(0, SPLIT)]],
                gbuf.at[s, pl.ds(0, SPLIT)], gsem.at[s]).wait()
            pltpu.make_async_copy(
                lut_hbm.at[idx_v.at[i, pl.ds(SPLIT, N_TOK - SPLIT)]],
                gbuf.at[s, pl.ds(SPLIT, N_TOK - SPLIT)], gsem.at[s]).wait()

        def out_start(s, i):
            pltpu.async_copy(sbuf.at[s], out_hbm.at[b0 + i], osem.at[s])

        def out_wait(s, i):
            pltpu.make_async_copy(sbuf.at[s], out_hbm.at[b0 + i],
                                  osem.at[s]).wait()

        for s in range(NBUF):
            gather_start(s, s)

        def body(it, carry):
            i0 = it * NBUF
            for s in range(NBUF):
                i = i0 + s

                @pl.when(i >= NBUF)
                def _():
                    out_wait(s, i - NBUF)

                gather_wait(s, i)

                def srow(r, c):
                    for q in range(D_MODEL // 16):
                        sbuf[s, pl.ds(r * D_MODEL + q * 16, 16)] = (
                            gbuf[s, r, pl.ds(q * 16, 16)] * SCALE)
                    return c

                lax.fori_loop(0, N_TOK, srow, 0)

                @pl.when(i + NBUF < B_PER_W)
                def _():
                    gather_start(s, i + NBUF)

                out_start(s, i)
            return carry

        lax.fori_loop(0, B_PER_W // NBUF, body, 0)

        for s in range(NBUF):
            out_wait(s, B_PER_W - NBUF + s)

    return k(lut, x)


def kernel(x, lut):
    lut_rm = _tc_pack_lut(lut)
    dense = _sc_embed(lut_rm, _permute_idx(x))
    return _tc_finish(dense)
